# Initial kernel scaffold; baseline (speedup 1.0000x reference)
import functools

import jax
import jax.numpy as jnp
from jax import lax
from jax.experimental import pallas as pl
from jax.experimental.pallas import tpu as pltpu
from jax.experimental.pallas import tpu_sc as plsc

_NC, _NS = 2, 16
_NW = _NC * _NS
_CHUNK = 512


def _gather_body(idx_hbm, table_hbm, out_hbm, idx_v, rows_v, sem):
    c = lax.axis_index("c")
    s = lax.axis_index("s")
    wid = s * _NC + c
    n = idx_hbm.shape[0]
    b_per_w = n // _NW
    steps = b_per_w // _CHUNK

    def body(i, carry):
        base = wid * b_per_w + i * _CHUNK
        pltpu.sync_copy(idx_hbm.at[pl.ds(base, _CHUNK)], idx_v)
        pltpu.async_copy(table_hbm.at[idx_v], rows_v, sem).wait()
        pltpu.sync_copy(rows_v, out_hbm.at[pl.ds(base, _CHUNK)])
        return carry

    lax.fori_loop(0, steps, body, 0)


def _sc_gather(idx_flat, wemb):
    n = idx_flat.shape[0]
    h = wemb.shape[1]
    mesh = plsc.VectorSubcoreMesh(core_axis_name="c", subcore_axis_name="s")
    return pl.kernel(
        _gather_body,
        out_type=jax.ShapeDtypeStruct((n, h), jnp.float32),
        mesh=mesh,
        scratch_types=[
            pltpu.VMEM((_CHUNK,), jnp.int32),
            pltpu.VMEM((_CHUNK, h), jnp.float32),
            pltpu.SemaphoreType.DMA,
        ],
    )(idx_flat, wemb)


def _ln_body(w_ref, p_ref, g_ref, b_ref, o_ref):
    e = w_ref[...] + p_ref[...][None]
    u = jnp.mean(e, axis=-1, keepdims=True)
    s = jnp.mean((e - u) ** 2, axis=-1, keepdims=True)
    x = (e - u) * lax.rsqrt(s + 1e-12)
    o_ref[...] = g_ref[...] * x + b_ref[...]


def _tc_layernorm(emb, posemb, gamma, beta):
    batch, seq, h = emb.shape
    blk = 32
    grid = (batch // blk,)
    return pl.pallas_call(
        _ln_body,
        grid=grid,
        in_specs=[
            pl.BlockSpec((blk, seq, h), lambda i: (i, 0, 0)),
            pl.BlockSpec((seq, h), lambda i: (0, 0)),
            pl.BlockSpec((1, h), lambda i: (0, 0)),
            pl.BlockSpec((1, h), lambda i: (0, 0)),
        ],
        out_specs=pl.BlockSpec((blk, seq, h), lambda i: (i, 0, 0)),
        out_shape=jax.ShapeDtypeStruct((batch, seq, h), jnp.float32),
    )(emb, posemb, gamma.reshape(1, h), beta.reshape(1, h))


def kernel(inputni, wemb, posemb, gamma, beta):
    batch, seq = inputni.shape
    h = wemb.shape[1]
    idx_flat = inputni.reshape(batch * seq).astype(jnp.int32)
    emb_w = _sc_gather(idx_flat, wemb)
    return _tc_layernorm(emb_w.reshape(batch, seq, h), posemb, gamma, beta)


# speedup vs baseline: 2.1798x; 2.1798x over previous
import functools

import jax
import jax.numpy as jnp
from jax import lax
from jax.experimental import pallas as pl
from jax.experimental.pallas import tpu as pltpu
from jax.experimental.pallas import tpu_sc as plsc

_NC, _NS = 2, 16
_NW = _NC * _NS
_CHUNK = 512


def _gather_body(idx_hbm, table_hbm, out_hbm, idx_v, rows_v, sem):
    c = lax.axis_index("c")
    s = lax.axis_index("s")
    wid = s * _NC + c
    n = idx_hbm.shape[0]
    b_per_w = n // _NW
    steps = b_per_w // _CHUNK

    def body(i, carry):
        base = wid * b_per_w + i * _CHUNK
        pltpu.sync_copy(idx_hbm.at[pl.ds(base, _CHUNK)], idx_v)
        pltpu.async_copy(table_hbm.at[idx_v], rows_v, sem).wait()
        pltpu.sync_copy(rows_v, out_hbm.at[pl.ds(base, _CHUNK)])
        return carry

    lax.fori_loop(0, steps, body, 0)


def _sc_gather(idx_flat, wemb):
    n = idx_flat.shape[0]
    h = wemb.shape[1]
    mesh = plsc.VectorSubcoreMesh(core_axis_name="c", subcore_axis_name="s")
    return pl.kernel(
        _gather_body,
        out_type=jax.ShapeDtypeStruct((n, h), jnp.float32),
        mesh=mesh,
        scratch_types=[
            pltpu.VMEM((_CHUNK,), jnp.int32),
            pltpu.VMEM((_CHUNK, h), jnp.float32),
            pltpu.SemaphoreType.DMA,
        ],
        compiler_params=pltpu.CompilerParams(use_tc_tiling_on_sc=False),
    )(idx_flat, wemb)


def _ln_body(w_ref, p_ref, g_ref, b_ref, o_ref):
    e = w_ref[...] + p_ref[...][None]
    u = jnp.mean(e, axis=-1, keepdims=True)
    s = jnp.mean((e - u) ** 2, axis=-1, keepdims=True)
    x = (e - u) * lax.rsqrt(s + 1e-12)
    o_ref[...] = g_ref[...] * x + b_ref[...]


def _tc_layernorm(emb, posemb, gamma, beta):
    batch, seq, h = emb.shape
    blk = 32
    grid = (batch // blk,)
    return pl.pallas_call(
        _ln_body,
        grid=grid,
        in_specs=[
            pl.BlockSpec((blk, seq, h), lambda i: (i, 0, 0)),
            pl.BlockSpec((seq, h), lambda i: (0, 0)),
            pl.BlockSpec((1, h), lambda i: (0, 0)),
            pl.BlockSpec((1, h), lambda i: (0, 0)),
        ],
        out_specs=pl.BlockSpec((blk, seq, h), lambda i: (i, 0, 0)),
        out_shape=jax.ShapeDtypeStruct((batch, seq, h), jnp.float32),
    )(emb, posemb, gamma.reshape(1, h), beta.reshape(1, h))


def kernel(inputni, wemb, posemb, gamma, beta):
    batch, seq = inputni.shape
    h = wemb.shape[1]
    idx_flat = inputni.reshape(batch * seq).astype(jnp.int32)
    emb_w = _sc_gather(idx_flat, wemb)
    return _tc_layernorm(emb_w.reshape(batch, seq, h), posemb, gamma, beta)
